# baseline (speedup 1.0000x reference)
import functools

import jax
import jax.numpy as jnp
from jax import lax
from jax.experimental import pallas as pl
from jax.experimental.pallas import tpu as pltpu
from jax.experimental.pallas import tpu_sc as plsc

_P = 64
_K = 8
_N_IDX = _P * _K
_D = 2048
_NC = 2
_NS = 16
_NW = _NC * _NS
_L = 16
_CH = 16
_NBUF = 2


def _build_sc_gather(R):
    rows_per_w = R // _NW
    n_chunks = rows_per_w // _CH
    n_groups = _N_IDX // _L
    mesh = plsc.VectorSubcoreMesh(core_axis_name="c", subcore_axis_name="s")

    @functools.partial(
        pl.kernel,
        mesh=mesh,
        out_type=jax.ShapeDtypeStruct((R, _N_IDX), jnp.float32),
        scratch_types=[
            pltpu.VMEM((_N_IDX,), jnp.int32),
            pltpu.VMEM((_CH, _D), jnp.float32),
            pltpu.VMEM((_CH, _D), jnp.float32),
            pltpu.VMEM((_CH, _N_IDX), jnp.float32),
            pltpu.VMEM((_CH, _N_IDX), jnp.float32),
            pltpu.SemaphoreType.DMA,
            pltpu.SemaphoreType.DMA,
            pltpu.SemaphoreType.DMA,
            pltpu.SemaphoreType.DMA,
        ],
        compiler_params=pltpu.CompilerParams(needs_layout_passes=False),
    )
    def sc_gather(y_hbm, idx_hbm, out_hbm, idx_v, in0, in1, out0, out1,
                  isem0, isem1, osem0, osem1):
        cid = lax.axis_index("c")
        sid = lax.axis_index("s")
        wid = sid * _NC + cid
        base = wid * rows_per_w
        pltpu.sync_copy(idx_hbm, idx_v)

        ins = (in0, in1)
        outs = (out0, out1)
        isems = (isem0, isem1)
        osems = (osem0, osem1)

        def start_in(c, s):
            row0 = base + c * _CH
            pltpu.make_async_copy(
                y_hbm.at[pl.ds(row0, _CH)], ins[s], isems[s]
            ).start()

        def wait_in(c, s):
            row0 = base + c * _CH
            pltpu.make_async_copy(
                y_hbm.at[pl.ds(row0, _CH)], ins[s], isems[s]
            ).wait()

        def start_out(c, s):
            row0 = base + c * _CH
            pltpu.make_async_copy(
                outs[s], out_hbm.at[pl.ds(row0, _CH)], osems[s]
            ).start()

        def wait_out(c, s):
            row0 = base + c * _CH
            pltpu.make_async_copy(
                outs[s], out_hbm.at[pl.ds(row0, _CH)], osems[s]
            ).wait()

        cvecs = [idx_v[pl.ds(g * _L, _L)] for g in range(n_groups)]

        for s in range(_NBUF):
            start_in(s, s)

        def loop_body(t, carry):
            for s in range(_NBUF):
                c = _NBUF * t + s
                wait_in(c, s)

                @pl.when(c >= _NBUF)
                def _():
                    wait_out(c - _NBUF, s)

                def do_row(r):
                    rvec = jnp.full((_L,), 0, jnp.int32) + r
                    for g in range(n_groups):
                        vals = plsc.load_gather(ins[s], [rvec, cvecs[g]])
                        outs[s][r, pl.ds(g * _L, _L)] = vals

                plsc.parallel_loop(0, _CH, 1, unroll=16)(do_row)

                start_out(c, s)

                @pl.when(c + _NBUF < n_chunks)
                def _():
                    start_in(c + _NBUF, s)
            return carry

        lax.fori_loop(0, n_chunks // _NBUF, loop_body, 0)

        for s in range(_NBUF):
            wait_out(n_chunks - _NBUF + s, s)

    return sc_gather


def kernel(y, src_idx):
    B, T, D = y.shape
    R = B * T
    y2d = y.reshape(R, D)
    idx_flat = src_idx.reshape(-1).astype(jnp.int32)
    out = _build_sc_gather(R)(y2d, idx_flat)
    return out.reshape(B, T, _P, _K)


# speedup vs baseline: 1.0750x; 1.0750x over previous
import functools

import jax
import jax.numpy as jnp
from jax import lax
from jax.experimental import pallas as pl
from jax.experimental.pallas import tpu as pltpu
from jax.experimental.pallas import tpu_sc as plsc

_P = 64
_K = 8
_N_IDX = _P * _K
_D = 2048
_NC = 2
_NS = 16
_NW = _NC * _NS
_L = 16
_CH = 16
_NBUF = 2


def _build_sc_gather(R):
    rows_per_w = R // _NW
    n_chunks = rows_per_w // _CH
    n_groups = _N_IDX // _L
    mesh = plsc.VectorSubcoreMesh(core_axis_name="c", subcore_axis_name="s")

    @functools.partial(
        pl.kernel,
        mesh=mesh,
        out_type=jax.ShapeDtypeStruct((R, _N_IDX), jnp.float32),
        scratch_types=[
            pltpu.VMEM((_N_IDX,), jnp.int32),
            pltpu.VMEM((_CH, _D), jnp.float32),
            pltpu.VMEM((_CH, _D), jnp.float32),
            pltpu.VMEM((_CH, _N_IDX), jnp.float32),
            pltpu.VMEM((_CH, _N_IDX), jnp.float32),
            pltpu.SemaphoreType.DMA,
            pltpu.SemaphoreType.DMA,
            pltpu.SemaphoreType.DMA,
            pltpu.SemaphoreType.DMA,
        ],
        compiler_params=pltpu.CompilerParams(needs_layout_passes=False),
    )
    def sc_gather(y_hbm, idx_hbm, out_hbm, idx_v, in0, in1, out0, out1,
                  isem0, isem1, osem0, osem1):
        cid = lax.axis_index("c")
        sid = lax.axis_index("s")
        wid = sid * _NC + cid
        base = wid * rows_per_w
        pltpu.sync_copy(idx_hbm, idx_v)

        ins = (in0, in1)
        outs = (out0, out1)
        isems = (isem0, isem1)
        osems = (osem0, osem1)

        def start_in(c, s):
            row0 = base + c * _CH
            pltpu.make_async_copy(
                y_hbm.at[pl.ds(row0, _CH)], ins[s], isems[s]
            ).start()

        def wait_in(c, s):
            row0 = base + c * _CH
            pltpu.make_async_copy(
                y_hbm.at[pl.ds(row0, _CH)], ins[s], isems[s]
            ).wait()

        def start_out(c, s):
            row0 = base + c * _CH
            pltpu.make_async_copy(
                outs[s], out_hbm.at[pl.ds(row0, _CH)], osems[s]
            ).start()

        def wait_out(c, s):
            row0 = base + c * _CH
            pltpu.make_async_copy(
                outs[s], out_hbm.at[pl.ds(row0, _CH)], osems[s]
            ).wait()

        cvecs = [idx_v[pl.ds(g * _L, _L)] for g in range(n_groups)]

        for s in range(_NBUF):
            start_in(s, s)

        def loop_body(t, carry):
            for s in range(_NBUF):
                c = _NBUF * t + s
                wait_in(c, s)

                @pl.when(c >= _NBUF)
                def _():
                    wait_out(c - _NBUF, s)

                def do_row(r):
                    rvec = jnp.full((_L,), 0, jnp.int32) + r
                    for g in range(n_groups):
                        vals = plsc.load_gather(ins[s], [rvec, cvecs[g]])
                        outs[s][r, pl.ds(g * _L, _L)] = vals

                plsc.parallel_loop(0, _CH, 1, unroll=8)(do_row)

                start_out(c, s)

                @pl.when(c + _NBUF < n_chunks)
                def _():
                    start_in(c + _NBUF, s)
            return carry

        lax.fori_loop(0, n_chunks // _NBUF, loop_body, 0)

        for s in range(_NBUF):
            wait_out(n_chunks - _NBUF + s, s)

    return sc_gather


def kernel(y, src_idx):
    B, T, D = y.shape
    R = B * T
    y2d = y.reshape(R, D)
    idx_flat = src_idx.reshape(-1).astype(jnp.int32)
    out = _build_sc_gather(R)(y2d, idx_flat)
    return out.reshape(B, T, _P, _K)
